# baseline (speedup 1.0000x reference)
import numpy as np
import jax
import jax.numpy as jnp
from jax import lax
from jax.experimental import pallas as pl
from jax.experimental.pallas import tpu as pltpu
from jax.experimental.pallas import tpu_sc as plsc

FIELDS = 26
FS = 100
EMB = 32
N_DENSE = 13
B = 1024
WIDTH = FIELDS * (FS + 1)

LANES = 16
K = 2688
HB = 2800
NW = 32
XW = 2688
XW2 = XW // 2

_offs_np = np.full((XW,), K, dtype=np.int32)
for _i in range(FIELDS):
    _offs_np[_i * 101:_i * 101 + FS] = _i * FS

IL = 2


def _make_sc_body(nb):
    rows = nb // NW
    nchunk = rows // IL

    def sc_body(x_hbm, out_hbm, x_vs, hist_vs, in_sems, out_sems):
        nc = 2
        wid = lax.axis_index("s") * nc + lax.axis_index("c")
        base = wid * rows
        ones = jnp.full((LANES,), 1.0, jnp.float32)
        zeros = jnp.zeros((LANES,), jnp.float32)

        for p in range(2):
            for u in range(IL):
                pltpu.async_copy(x_hbm.at[base + p * IL + u], x_vs[p][u],
                                 in_sems[p][u])

        def do_phase(p, c):
            b = base + c * IL

            @pl.when(c >= 2)
            def _wait_hist_free():
                for u in range(IL):
                    pltpu.make_async_copy(hist_vs[p][u].at[pl.ds(0, K)],
                                          out_hbm.at[b - 2 * IL + u],
                                          out_sems[p][u]).wait()

            for g in range(HB // LANES):
                for u in range(IL):
                    hist_vs[p][u][pl.ds(g * LANES, LANES)] = zeros
            for u in range(IL):
                pltpu.make_async_copy(x_hbm.at[b + u], x_vs[p][u],
                                      in_sems[p][u]).wait()
            for g in range(XW2 // LANES):
                for u in range(IL):
                    vp = x_vs[p][u][pl.ds(g * LANES, LANES)]
                    va = lax.bitwise_and(vp, jnp.int32(0xFFFF))
                    vb = lax.shift_right_logical(vp, jnp.int32(16))
                    plsc.addupdate_scatter(hist_vs[p][u], [va], ones)
                    plsc.addupdate_scatter(hist_vs[p][u], [vb], ones)

            @pl.when(c + 2 < nchunk)
            def _prefetch():
                for u in range(IL):
                    pltpu.async_copy(x_hbm.at[b + 2 * IL + u], x_vs[p][u],
                                     in_sems[p][u])

            for u in range(IL):
                pltpu.async_copy(hist_vs[p][u].at[pl.ds(0, K)],
                                 out_hbm.at[b + u], out_sems[p][u])

        def body(it, carry):
            do_phase(0, 2 * it)
            do_phase(1, 2 * it + 1)
            return carry

        lax.fori_loop(0, nchunk // 2, body, 0)
        last = base + (nchunk - 2) * IL
        for p in range(2):
            for u in range(IL):
                pltpu.make_async_copy(hist_vs[p][u].at[pl.ds(0, K)],
                                      out_hbm.at[last + p * IL + u],
                                      out_sems[p][u]).wait()

    return sc_body


def _histogram(xp):
    nb = xp.shape[0]
    mesh = plsc.VectorSubcoreMesh(core_axis_name="c", subcore_axis_name="s")
    return pl.kernel(
        _make_sc_body(nb),
        out_type=jax.ShapeDtypeStruct((nb, K), jnp.float32),
        mesh=mesh,
        scratch_types=[
            [[pltpu.VMEM((XW2,), jnp.int32) for _ in range(IL)]
             for _ in range(2)],
            [[pltpu.VMEM((HB,), jnp.float32) for _ in range(IL)]
             for _ in range(2)],
            [[pltpu.SemaphoreType.DMA for _ in range(IL)] for _ in range(2)],
            [[pltpu.SemaphoreType.DMA for _ in range(IL)] for _ in range(2)],
        ],
        compiler_params=pltpu.CompilerParams(needs_layout_passes=False),
    )(xp)


def _tc_fm_body(c_ref, t_ref, dense_ref, wd_ref, b_ref, o_ref):
    c = c_ref[...]
    t = t_ref[...]
    r1 = jnp.dot(c, t, preferred_element_type=jnp.float32)
    r2 = jnp.dot(c, t * t, preferred_element_type=jnp.float32)
    s = r1[:, :EMB]
    lin = r1[:, EMB:EMB + 1]
    ss = jnp.sum(r2[:, :EMB], axis=1, keepdims=True)
    cross = 0.5 * (jnp.sum(s * s, axis=1, keepdims=True) - ss)
    dlin = jnp.dot(dense_ref[...], wd_ref[...],
                   preferred_element_type=jnp.float32)
    o_ref[...] = jax.nn.sigmoid(lin + dlin + b_ref[0, 0] + cross)


def _fm_tail(counts, table, dense_feat, w_dense, bias):
    nb = counts.shape[0]
    blk = 256
    return pl.pallas_call(
        _tc_fm_body,
        grid=(nb // blk,),
        in_specs=[
            pl.BlockSpec((blk, K), lambda i: (i, 0)),
            pl.BlockSpec((K, 64), lambda i: (0, 0)),
            pl.BlockSpec((blk, N_DENSE), lambda i: (i, 0)),
            pl.BlockSpec((N_DENSE, 1), lambda i: (0, 0)),
            pl.BlockSpec((1, 1), lambda i: (0, 0)),
        ],
        out_specs=pl.BlockSpec((blk, 1), lambda i: (i, 0)),
        out_shape=jax.ShapeDtypeStruct((nb, 1), jnp.float32),
    )(counts, table, dense_feat, w_dense, bias)


@jax.jit
def kernel(sparse_feat, dense_feat, linear_emb, emb, lin_W, lin_b):
    x = sparse_feat.astype(jnp.int32)
    xb = jnp.pad(x, ((0, 0), (0, XW - WIDTH))) + jnp.asarray(_offs_np)
    xp = xb[:, :XW2] | (xb[:, XW2:] << 16)

    emb_flat = emb.reshape(FIELDS * FS, EMB)
    lvec = (linear_emb[:, :, 0] * lin_W[:FIELDS, 0:1]).reshape(FIELDS * FS, 1)
    table = jnp.concatenate(
        [emb_flat, lvec, jnp.zeros((FIELDS * FS, 64 - EMB - 1), jnp.float32)],
        axis=1)
    table = jnp.pad(table, ((0, K - FIELDS * FS), (0, 0)))
    w_dense = lin_W[FIELDS:]
    bias = lin_b.reshape(1, 1)

    H = B // 2
    outs = []
    counts = [_histogram(xp[h * H:(h + 1) * H]) for h in range(2)]
    for h in range(2):
        outs.append(_fm_tail(counts[h], table,
                             dense_feat[h * H:(h + 1) * H], w_dense, bias))
    return jnp.concatenate(outs, axis=0)


# speedup vs baseline: 1.0709x; 1.0709x over previous
import numpy as np
import jax
import jax.numpy as jnp
from jax import lax
from jax.experimental import pallas as pl
from jax.experimental.pallas import tpu as pltpu
from jax.experimental.pallas import tpu_sc as plsc

FIELDS = 26
FS = 100
EMB = 32
N_DENSE = 13
B = 1024
WIDTH = FIELDS * (FS + 1)

LANES = 16
K = 2688
HB = 2800
NW = 32
XW = 2688
XW2 = XW // 2

_offs_np = np.full((XW,), K, dtype=np.int32)
for _i in range(FIELDS):
    _offs_np[_i * 101:_i * 101 + FS] = _i * FS

IL = 2


def _make_sc_body(nb):
    rows = nb // NW
    nchunk = rows // IL

    def sc_body(x_hbm, out_hbm, x_vs, hist_vs, in_sems, out_sems):
        nc = 2
        wid = lax.axis_index("s") * nc + lax.axis_index("c")
        base = wid * rows
        ones = jnp.full((LANES,), 1.0, jnp.float32)
        zeros = jnp.zeros((LANES,), jnp.float32)

        for p in range(2):
            for u in range(IL):
                pltpu.async_copy(x_hbm.at[base + p * IL + u], x_vs[p][u],
                                 in_sems[p][u])

        def do_phase(p, c):
            b = base + c * IL

            @pl.when(c >= 2)
            def _wait_hist_free():
                for u in range(IL):
                    pltpu.make_async_copy(hist_vs[p][u].at[pl.ds(0, K)],
                                          out_hbm.at[b - 2 * IL + u],
                                          out_sems[p][u]).wait()

            for g in range(HB // LANES):
                for u in range(IL):
                    hist_vs[p][u][pl.ds(g * LANES, LANES)] = zeros
            for u in range(IL):
                pltpu.make_async_copy(x_hbm.at[b + u], x_vs[p][u],
                                      in_sems[p][u]).wait()
            for g in range(XW2 // LANES):
                for u in range(IL):
                    vp = x_vs[p][u][pl.ds(g * LANES, LANES)]
                    va = lax.bitwise_and(vp, jnp.int32(0xFFFF))
                    vb = lax.shift_right_logical(vp, jnp.int32(16))
                    plsc.addupdate_scatter(hist_vs[p][u], [va], ones)
                    plsc.addupdate_scatter(hist_vs[p][u], [vb], ones)

            @pl.when(c + 2 < nchunk)
            def _prefetch():
                for u in range(IL):
                    pltpu.async_copy(x_hbm.at[b + 2 * IL + u], x_vs[p][u],
                                     in_sems[p][u])

            for u in range(IL):
                pltpu.async_copy(hist_vs[p][u].at[pl.ds(0, K)],
                                 out_hbm.at[b + u], out_sems[p][u])

        def body(it, carry):
            do_phase(0, 2 * it)
            do_phase(1, 2 * it + 1)
            return carry

        lax.fori_loop(0, nchunk // 2, body, 0)
        last = base + (nchunk - 2) * IL
        for p in range(2):
            for u in range(IL):
                pltpu.make_async_copy(hist_vs[p][u].at[pl.ds(0, K)],
                                      out_hbm.at[last + p * IL + u],
                                      out_sems[p][u]).wait()

    return sc_body


def _histogram(xp):
    nb = xp.shape[0]
    mesh = plsc.VectorSubcoreMesh(core_axis_name="c", subcore_axis_name="s")
    return pl.kernel(
        _make_sc_body(nb),
        out_type=jax.ShapeDtypeStruct((nb, K), jnp.float32),
        mesh=mesh,
        scratch_types=[
            [[pltpu.VMEM((XW2,), jnp.int32) for _ in range(IL)]
             for _ in range(2)],
            [[pltpu.VMEM((HB,), jnp.float32) for _ in range(IL)]
             for _ in range(2)],
            [[pltpu.SemaphoreType.DMA for _ in range(IL)] for _ in range(2)],
            [[pltpu.SemaphoreType.DMA for _ in range(IL)] for _ in range(2)],
        ],
        compiler_params=pltpu.CompilerParams(needs_layout_passes=False),
    )(xp)


def _tc_fm_body(c_ref, t_ref, dense_ref, wd_ref, b_ref, o_ref):
    c = c_ref[...]
    t = t_ref[...]
    r1 = jnp.dot(c, t, preferred_element_type=jnp.float32)
    r2 = jnp.dot(c, t * t, preferred_element_type=jnp.float32)
    s = r1[:, :EMB]
    lin = r1[:, EMB:EMB + 1]
    ss = jnp.sum(r2[:, :EMB], axis=1, keepdims=True)
    cross = 0.5 * (jnp.sum(s * s, axis=1, keepdims=True) - ss)
    dlin = jnp.dot(dense_ref[...], wd_ref[...],
                   preferred_element_type=jnp.float32)
    o_ref[...] = jax.nn.sigmoid(lin + dlin + b_ref[0, 0] + cross)


def _fm_tail(counts, table, dense_feat, w_dense, bias):
    nb = counts.shape[0]
    blk = 256
    return pl.pallas_call(
        _tc_fm_body,
        grid=(nb // blk,),
        in_specs=[
            pl.BlockSpec((blk, K), lambda i: (i, 0)),
            pl.BlockSpec((K, 64), lambda i: (0, 0)),
            pl.BlockSpec((blk, N_DENSE), lambda i: (i, 0)),
            pl.BlockSpec((N_DENSE, 1), lambda i: (0, 0)),
            pl.BlockSpec((1, 1), lambda i: (0, 0)),
        ],
        out_specs=pl.BlockSpec((blk, 1), lambda i: (i, 0)),
        out_shape=jax.ShapeDtypeStruct((nb, 1), jnp.float32),
    )(counts, table, dense_feat, w_dense, bias)


@jax.jit
def kernel(sparse_feat, dense_feat, linear_emb, emb, lin_W, lin_b):
    x = sparse_feat.astype(jnp.int32)
    xb = jnp.pad(x, ((0, 0), (0, XW - WIDTH))) + jnp.asarray(_offs_np)
    xp = xb[:, :XW2] | (xb[:, XW2:] << 16)

    emb_flat = emb.reshape(FIELDS * FS, EMB)
    lvec = (linear_emb[:, :, 0] * lin_W[:FIELDS, 0:1]).reshape(FIELDS * FS, 1)
    table = jnp.concatenate(
        [emb_flat, lvec, jnp.zeros((FIELDS * FS, 64 - EMB - 1), jnp.float32)],
        axis=1)
    table = jnp.pad(table, ((0, K - FIELDS * FS), (0, 0)))
    w_dense = lin_W[FIELDS:]
    bias = lin_b.reshape(1, 1)

    counts = _histogram(xp)
    return _fm_tail(counts, table, dense_feat, w_dense, bias)
